# baseline (speedup 1.0000x reference)
import functools

import jax
import jax.numpy as jnp
from jax import lax
from jax.experimental import pallas as pl
from jax.experimental.pallas import tpu as pltpu
from jax.experimental.pallas import tpu_sc as plsc

HIDDEN = 1024
INTER = 1408
NUM_EXPERTS = 8
TOP_K = 2
T = 2048
EPAD = 128

BM = 128
NBLK = (T * TOP_K + NUM_EXPERTS * BM) // BM
PADTOT = NBLK * BM
NTILE = 32
NCH = 4
SCH = T * TOP_K // NTILE // NCH
TPW = T // NTILE


def _router_body(x_ref, gw_ref, pos_ref, gate_ref, bexp_ref):
    xf = x_ref[...]
    logits = lax.dot_general(xf, gw_ref[...], (((1,), (1,)), ((), ())),
                             preferred_element_type=jnp.float32)
    col = lax.broadcasted_iota(jnp.int32, (T, EPAD), 1)
    valid = col < NUM_EXPERTS
    logits = jnp.where(valid, logits, -jnp.inf)
    m = jnp.max(logits, axis=1, keepdims=True)
    p = jnp.exp(logits - m)
    probs = p / jnp.sum(p, axis=1, keepdims=True)
    g1 = jnp.max(probs, axis=1, keepdims=True)
    e1 = jnp.min(jnp.where(probs == g1, col, EPAD), axis=1, keepdims=True)
    probs2 = jnp.where(col == e1, -1.0, probs)
    g2 = jnp.max(probs2, axis=1, keepdims=True)
    e2 = jnp.min(jnp.where(probs2 == g2, col, EPAD), axis=1, keepdims=True)
    gsum = jnp.clip(g1 + g2, 1e-12, None)
    gate_ref[:T, :] = jnp.broadcast_to(g1 / gsum, (T, EPAD))
    gate_ref[T:, :] = jnp.broadcast_to(g2 / gsum, (T, EPAD))

    m0 = col == e1
    m1 = col == e2
    c01 = jnp.where(jnp.logical_or(m0, m1), 1.0, 0.0)
    it = lax.broadcasted_iota(jnp.int32, (T, T), 0)
    jt = lax.broadcasted_iota(jnp.int32, (T, T), 1)
    ltri = jnp.where(it > jt, 1.0, 0.0).astype(jnp.bfloat16)
    s = lax.dot_general(ltri, c01.astype(jnp.bfloat16),
                        (((1,), (0,)), ((), ())),
                        preferred_element_type=jnp.float32)
    counts = jnp.sum(c01, axis=0, keepdims=True)
    pc = jnp.floor((counts + (BM - 1)) * (1.0 / BM)) * BM
    ie = lax.broadcasted_iota(jnp.int32, (EPAD, EPAD), 0)
    je = lax.broadcasted_iota(jnp.int32, (EPAD, EPAD), 1)
    ustr = jnp.where(ie < je, 1.0, 0.0).astype(jnp.bfloat16)
    off = lax.dot_general(pc.astype(jnp.bfloat16), ustr,
                          (((1,), (0,)), ((), ())),
                          preferred_element_type=jnp.float32)
    offb = jnp.broadcast_to(off, (T, EPAD))
    pos0 = jnp.sum(jnp.where(m0, offb + s, 0.0), axis=1, keepdims=True)
    pos1 = jnp.sum(jnp.where(m1, offb + s, 0.0), axis=1, keepdims=True)
    pos_ref[:, 0:1] = pos0.astype(jnp.int32)
    pos_ref[:, 1:2] = pos1.astype(jnp.int32)

    mi = lax.broadcasted_iota(jnp.int32, (EPAD, EPAD), 0)
    offr = jnp.broadcast_to(off, (EPAD, EPAD))
    a = jnp.where(jnp.logical_and((mi * BM).astype(jnp.float32) >= offr,
                                  je < NUM_EXPERTS), 1.0, 0.0)
    bexp = jnp.sum(a, axis=1, keepdims=True) - 1.0
    bexp_ref[...] = bexp.astype(jnp.int32)


def _router(x, gwp):
    return pl.pallas_call(
        _router_body,
        out_shape=(
            jax.ShapeDtypeStruct((T, 2), jnp.int32),
            jax.ShapeDtypeStruct((T * TOP_K, EPAD), jnp.float32),
            jax.ShapeDtypeStruct((EPAD, 1), jnp.int32),
        ),
    )(x, gwp)


def _dispatch(x, pos3, g16):
    mesh = plsc.VectorSubcoreMesh(core_axis_name="c", subcore_axis_name="s")

    @functools.partial(
        pl.kernel,
        out_type=(
            jax.ShapeDtypeStruct((PADTOT, HIDDEN), jnp.float32),
            jax.ShapeDtypeStruct((PADTOT, 128), jnp.float32),
        ),
        mesh=mesh,
        scratch_types=[
            pltpu.VMEM((NCH, SCH), jnp.int32),
            pltpu.VMEM((SCH, HIDDEN), jnp.float32),
            pltpu.VMEM((SCH, HIDDEN), jnp.float32),
            pltpu.VMEM((SCH, 128), jnp.float32),
            pltpu.VMEM((SCH, 128), jnp.float32),
            pltpu.SemaphoreType.DMA,
            pltpu.SemaphoreType.DMA,
            pltpu.SemaphoreType.DMA,
            pltpu.SemaphoreType.DMA,
            pltpu.SemaphoreType.DMA,
            pltpu.SemaphoreType.DMA,
        ],
    )
    def k(x_hbm, pos3_hbm, g16_hbm, xs_hbm, gs_hbm,
          idx_v, rows_a, rows_b, grows_a, grows_b,
          sxa, sxb, sga, sgb, ssx, ssg):
        wid = lax.axis_index("s") * 2 + lax.axis_index("c")
        pltpu.sync_copy(pos3_hbm.at[wid], idx_v)
        tok0 = lax.rem(wid, 16) * (NCH * SCH)
        flat0 = wid * (NCH * SCH)
        xbuf = (rows_a, rows_b)
        gbuf = (grows_a, grows_b)
        xsem = (sxa, sxb)
        gsem = (sga, sgb)
        lds = {}
        for j in range(2):
            lds[j] = (
                pltpu.async_copy(x_hbm.at[pl.ds(tok0 + j * SCH, SCH)],
                                 xbuf[j], xsem[j]),
                pltpu.async_copy(g16_hbm.at[pl.ds(flat0 + j * SCH, SCH)],
                                 gbuf[j], gsem[j]),
            )
        scs = []
        for j in range(NCH):
            p = j % 2
            lx, lg = lds[j]
            lx.wait()
            scs.append(pltpu.async_copy(xbuf[p], xs_hbm.at[idx_v.at[j]],
                                        ssx))
            lg.wait()
            scs.append(pltpu.async_copy(gbuf[p], gs_hbm.at[idx_v.at[j]],
                                        ssg))
            if j + 2 < NCH:
                scs[-2].wait()
                scs[-1].wait()
                scs.pop()
                scs.pop()
                lds[j + 2] = (
                    pltpu.async_copy(
                        x_hbm.at[pl.ds(tok0 + (j + 2) * SCH, SCH)],
                        xbuf[p], xsem[p]),
                    pltpu.async_copy(
                        g16_hbm.at[pl.ds(flat0 + (j + 2) * SCH, SCH)],
                        gbuf[p], gsem[p]),
                )
        for c in scs:
            c.wait()

    return k(x, pos3, g16)


def _ggemm_body(bexp_ref, xs_ref, w1a_ref, w1b_ref, w2_ref, gs_ref, ys_ref,
                w1a_s, w1b_s, w2_s):
    m = pl.program_id(0)
    prev = bexp_ref[jnp.maximum(m - 1, 0)]
    changed = jnp.logical_or(m == 0, bexp_ref[m] != prev)

    @pl.when(changed)
    def _cast():
        w1a_s[...] = w1a_ref[0].astype(jnp.bfloat16)
        w1b_s[...] = w1b_ref[0].astype(jnp.bfloat16)
        w2_s[...] = w2_ref[0].astype(jnp.bfloat16)

    xb = xs_ref[...].astype(jnp.bfloat16)
    a = lax.dot_general(xb, w1a_s[...], (((1,), (1,)), ((), ())),
                        preferred_element_type=jnp.float32)
    b = lax.dot_general(xb, w1b_s[...], (((1,), (1,)), ((), ())),
                        preferred_element_type=jnp.float32)
    act = (a * jax.nn.sigmoid(a) * b).astype(jnp.bfloat16)
    y = lax.dot_general(act, w2_s[...], (((1,), (1,)), ((), ())),
                        preferred_element_type=jnp.float32)
    ys_ref[...] = y * gs_ref[...][:, 0:1]


def _ggemm(bexp, xs, gs, w1, w2):
    grid_spec = pltpu.PrefetchScalarGridSpec(
        num_scalar_prefetch=1,
        grid=(NBLK,),
        in_specs=[
            pl.BlockSpec((BM, HIDDEN), lambda m, b: (m, 0)),
            pl.BlockSpec((1, INTER, HIDDEN), lambda m, b: (b[m], 0, 0)),
            pl.BlockSpec((1, INTER, HIDDEN), lambda m, b: (b[m], 1, 0)),
            pl.BlockSpec((1, HIDDEN, INTER), lambda m, b: (b[m], 0, 0)),
            pl.BlockSpec((BM, 128), lambda m, b: (m, 0)),
        ],
        out_specs=pl.BlockSpec((BM, HIDDEN), lambda m, b: (m, 0)),
        scratch_shapes=[
            pltpu.VMEM((INTER, HIDDEN), jnp.bfloat16),
            pltpu.VMEM((INTER, HIDDEN), jnp.bfloat16),
            pltpu.VMEM((HIDDEN, INTER), jnp.bfloat16),
        ],
    )
    return pl.pallas_call(
        _ggemm_body,
        grid_spec=grid_spec,
        out_shape=jax.ShapeDtypeStruct((PADTOT, HIDDEN), jnp.float32),
        compiler_params=pltpu.CompilerParams(
            dimension_semantics=("arbitrary",)),
    )(bexp, xs, w1, w1, w2, gs)


def _combine(ys, posf):
    mesh = plsc.VectorSubcoreMesh(core_axis_name="c", subcore_axis_name="s")

    ch = TPW // 2

    @functools.partial(
        pl.kernel,
        out_type=jax.ShapeDtypeStruct((T, HIDDEN), jnp.float32),
        mesh=mesh,
        scratch_types=[
            pltpu.VMEM((TPW,), jnp.int32),
            pltpu.VMEM((TPW,), jnp.int32),
            pltpu.VMEM((ch, HIDDEN), jnp.float32),
            pltpu.VMEM((ch, HIDDEN), jnp.float32),
            pltpu.VMEM((ch, HIDDEN), jnp.float32),
            pltpu.SemaphoreType.DMA,
            pltpu.SemaphoreType.DMA,
            pltpu.SemaphoreType.DMA,
            pltpu.SemaphoreType.DMA,
        ],
    )
    def k(ys_hbm, posf_hbm, out_hbm, idx0_v, idx1_v, bufa, bufb, bufc,
          sa, sb, sc, so):
        wid = lax.axis_index("s") * 2 + lax.axis_index("c")
        base = wid * TPW
        pltpu.sync_copy(posf_hbm.at[pl.ds(base, TPW)], idx0_v)
        pltpu.sync_copy(posf_hbm.at[pl.ds(T + base, TPW)], idx1_v)

        def add_into(dst, src):
            def arow(i, _):
                def agrp(g, _):
                    plsc.addupdate(dst.at[i, pl.ds(g * 16, 16)],
                                   src[i, pl.ds(g * 16, 16)])
                    return 0
                lax.fori_loop(0, HIDDEN // 16, agrp, 0, unroll=8)
                return 0
            lax.fori_loop(0, ch, arow, 0)

        ca = pltpu.async_copy(ys_hbm.at[idx0_v.at[pl.ds(0, ch)]], bufa, sa)
        cb = pltpu.async_copy(ys_hbm.at[idx1_v.at[pl.ds(0, ch)]], bufb, sb)
        cc = pltpu.async_copy(ys_hbm.at[idx0_v.at[pl.ds(ch, ch)]], bufc, sc)
        ca.wait()
        cb.wait()
        add_into(bufa, bufb)
        st_a = pltpu.async_copy(bufa, out_hbm.at[pl.ds(base, ch)], so)
        cb2 = pltpu.async_copy(ys_hbm.at[idx1_v.at[pl.ds(ch, ch)]], bufb, sb)
        cc.wait()
        cb2.wait()
        add_into(bufc, bufb)
        st_c = pltpu.async_copy(bufc, out_hbm.at[pl.ds(base + ch, ch)], so)
        st_a.wait()
        st_c.wait()

    return k(ys, posf)


def kernel(hidden_states, gate_weight, w1, w2):
    x = hidden_states.reshape(T, HIDDEN)
    gwp = jnp.zeros((EPAD, HIDDEN), jnp.float32).at[:NUM_EXPERTS].set(
        gate_weight)

    pos2, g16, bexp_col = _router(x, gwp)
    posf = pos2.T.reshape(T * TOP_K)
    pos3 = posf.reshape(NTILE, NCH, SCH)
    bexp = bexp_col[:NBLK, 0]

    xs, gs = _dispatch(x, pos3, g16)
    ys = _ggemm(bexp, xs, gs, w1, w2)
    out = _combine(ys, posf)
    return out.reshape(1, T, HIDDEN)


# speedup vs baseline: 1.2700x; 1.2700x over previous
import functools

import jax
import jax.numpy as jnp
from jax import lax
from jax.experimental import pallas as pl
from jax.experimental.pallas import tpu as pltpu
from jax.experimental.pallas import tpu_sc as plsc

HIDDEN = 1024
INTER = 1408
NUM_EXPERTS = 8
TOP_K = 2
T = 2048
EPAD = 128

BM = 256
NBLK = (T * TOP_K + NUM_EXPERTS * BM) // BM
PADTOT = NBLK * BM
NTILE = 32
NCH = 4
SCH = T * TOP_K // NTILE // NCH
TPW = T // NTILE


def _router_body(x_ref, gw_ref, pos_ref, gate_ref, bexp_ref):
    xf = x_ref[...]
    logits = lax.dot_general(xf, gw_ref[...], (((1,), (1,)), ((), ())),
                             preferred_element_type=jnp.float32)
    col = lax.broadcasted_iota(jnp.int32, (T, EPAD), 1)
    valid = col < NUM_EXPERTS
    logits = jnp.where(valid, logits, -jnp.inf)
    m = jnp.max(logits, axis=1, keepdims=True)
    p = jnp.exp(logits - m)
    probs = p / jnp.sum(p, axis=1, keepdims=True)
    g1 = jnp.max(probs, axis=1, keepdims=True)
    e1 = jnp.min(jnp.where(probs == g1, col, EPAD), axis=1, keepdims=True)
    probs2 = jnp.where(col == e1, -1.0, probs)
    g2 = jnp.max(probs2, axis=1, keepdims=True)
    e2 = jnp.min(jnp.where(probs2 == g2, col, EPAD), axis=1, keepdims=True)
    gsum = jnp.clip(g1 + g2, 1e-12, None)
    gate_ref[:T, :] = jnp.broadcast_to(g1 / gsum, (T, EPAD))
    gate_ref[T:, :] = jnp.broadcast_to(g2 / gsum, (T, EPAD))

    m0 = col == e1
    m1 = col == e2
    c01 = jnp.where(jnp.logical_or(m0, m1), 1.0, 0.0)
    it = lax.broadcasted_iota(jnp.int32, (T, T), 0)
    jt = lax.broadcasted_iota(jnp.int32, (T, T), 1)
    ltri = jnp.where(it > jt, 1.0, 0.0).astype(jnp.bfloat16)
    s = lax.dot_general(ltri, c01.astype(jnp.bfloat16),
                        (((1,), (0,)), ((), ())),
                        preferred_element_type=jnp.float32)
    counts = jnp.sum(c01, axis=0, keepdims=True)
    pc = jnp.floor((counts + (BM - 1)) * (1.0 / BM)) * BM
    ie = lax.broadcasted_iota(jnp.int32, (EPAD, EPAD), 0)
    je = lax.broadcasted_iota(jnp.int32, (EPAD, EPAD), 1)
    ustr = jnp.where(ie < je, 1.0, 0.0).astype(jnp.bfloat16)
    off = lax.dot_general(pc.astype(jnp.bfloat16), ustr,
                          (((1,), (0,)), ((), ())),
                          preferred_element_type=jnp.float32)
    offb = jnp.broadcast_to(off, (T, EPAD))
    pos0 = jnp.sum(jnp.where(m0, offb + s, 0.0), axis=1, keepdims=True)
    pos1 = jnp.sum(jnp.where(m1, offb + s, 0.0), axis=1, keepdims=True)
    pos_ref[:, 0:1] = pos0.astype(jnp.int32)
    pos_ref[:, 1:2] = pos1.astype(jnp.int32)

    mi = lax.broadcasted_iota(jnp.int32, (EPAD, EPAD), 0)
    offr = jnp.broadcast_to(off, (EPAD, EPAD))
    a = jnp.where(jnp.logical_and((mi * BM).astype(jnp.float32) >= offr,
                                  je < NUM_EXPERTS), 1.0, 0.0)
    bexp = jnp.sum(a, axis=1, keepdims=True) - 1.0
    bexp_ref[...] = bexp.astype(jnp.int32)


def _router(x, gwp):
    return pl.pallas_call(
        _router_body,
        out_shape=(
            jax.ShapeDtypeStruct((T, 2), jnp.int32),
            jax.ShapeDtypeStruct((T * TOP_K, EPAD), jnp.float32),
            jax.ShapeDtypeStruct((EPAD, 1), jnp.int32),
        ),
    )(x, gwp)


def _dispatch(x, pos3, g16):
    mesh = plsc.VectorSubcoreMesh(core_axis_name="c", subcore_axis_name="s")

    @functools.partial(
        pl.kernel,
        out_type=(
            jax.ShapeDtypeStruct((PADTOT, HIDDEN), jnp.float32),
            jax.ShapeDtypeStruct((PADTOT, 128), jnp.float32),
        ),
        mesh=mesh,
        scratch_types=[
            pltpu.VMEM((NCH, SCH), jnp.int32),
            pltpu.VMEM((SCH, HIDDEN), jnp.float32),
            pltpu.VMEM((SCH, HIDDEN), jnp.float32),
            pltpu.VMEM((SCH, 128), jnp.float32),
            pltpu.VMEM((SCH, 128), jnp.float32),
            pltpu.SemaphoreType.DMA,
            pltpu.SemaphoreType.DMA,
            pltpu.SemaphoreType.DMA,
            pltpu.SemaphoreType.DMA,
            pltpu.SemaphoreType.DMA,
            pltpu.SemaphoreType.DMA,
        ],
    )
    def k(x_hbm, pos3_hbm, g16_hbm, xs_hbm, gs_hbm,
          idx_v, rows_a, rows_b, grows_a, grows_b,
          sxa, sxb, sga, sgb, ssx, ssg):
        wid = lax.axis_index("s") * 2 + lax.axis_index("c")
        pltpu.sync_copy(pos3_hbm.at[wid], idx_v)
        tok0 = lax.rem(wid, 16) * (NCH * SCH)
        flat0 = wid * (NCH * SCH)
        xbuf = (rows_a, rows_b)
        gbuf = (grows_a, grows_b)
        xsem = (sxa, sxb)
        gsem = (sga, sgb)
        lds = {}
        for j in range(2):
            lds[j] = (
                pltpu.async_copy(x_hbm.at[pl.ds(tok0 + j * SCH, SCH)],
                                 xbuf[j], xsem[j]),
                pltpu.async_copy(g16_hbm.at[pl.ds(flat0 + j * SCH, SCH)],
                                 gbuf[j], gsem[j]),
            )
        scs = []
        for j in range(NCH):
            p = j % 2
            lx, lg = lds[j]
            lx.wait()
            scs.append(pltpu.async_copy(xbuf[p], xs_hbm.at[idx_v.at[j]],
                                        ssx))
            lg.wait()
            scs.append(pltpu.async_copy(gbuf[p], gs_hbm.at[idx_v.at[j]],
                                        ssg))
            if j + 2 < NCH:
                scs[-2].wait()
                scs[-1].wait()
                scs.pop()
                scs.pop()
                lds[j + 2] = (
                    pltpu.async_copy(
                        x_hbm.at[pl.ds(tok0 + (j + 2) * SCH, SCH)],
                        xbuf[p], xsem[p]),
                    pltpu.async_copy(
                        g16_hbm.at[pl.ds(flat0 + (j + 2) * SCH, SCH)],
                        gbuf[p], gsem[p]),
                )
        for c in scs:
            c.wait()

    return k(x, pos3, g16)


def _ggemm_body(bexp_ref, xs_ref, w1a_ref, w1b_ref, w2_ref, gs_ref, ys_ref,
                w1a_s, w1b_s, w2_s):
    m = pl.program_id(0)
    prev = bexp_ref[jnp.maximum(m - 1, 0)]
    changed = jnp.logical_or(m == 0, bexp_ref[m] != prev)

    @pl.when(changed)
    def _cast():
        w1a_s[...] = w1a_ref[0].astype(jnp.bfloat16)
        w1b_s[...] = w1b_ref[0].astype(jnp.bfloat16)
        w2_s[...] = w2_ref[0].astype(jnp.bfloat16)

    xb = xs_ref[...].astype(jnp.bfloat16)
    a = lax.dot_general(xb, w1a_s[...], (((1,), (1,)), ((), ())),
                        preferred_element_type=jnp.float32)
    b = lax.dot_general(xb, w1b_s[...], (((1,), (1,)), ((), ())),
                        preferred_element_type=jnp.float32)
    act = (a * jax.nn.sigmoid(a) * b).astype(jnp.bfloat16)
    y = lax.dot_general(act, w2_s[...], (((1,), (1,)), ((), ())),
                        preferred_element_type=jnp.float32)
    ys_ref[...] = y * gs_ref[...][:, 0:1]


def _ggemm(bexp, xs, gs, w1, w2):
    grid_spec = pltpu.PrefetchScalarGridSpec(
        num_scalar_prefetch=1,
        grid=(NBLK,),
        in_specs=[
            pl.BlockSpec((BM, HIDDEN), lambda m, b: (m, 0)),
            pl.BlockSpec((1, INTER, HIDDEN), lambda m, b: (b[m], 0, 0)),
            pl.BlockSpec((1, INTER, HIDDEN), lambda m, b: (b[m], 1, 0)),
            pl.BlockSpec((1, HIDDEN, INTER), lambda m, b: (b[m], 0, 0)),
            pl.BlockSpec((BM, 128), lambda m, b: (m, 0)),
        ],
        out_specs=pl.BlockSpec((BM, HIDDEN), lambda m, b: (m, 0)),
        scratch_shapes=[
            pltpu.VMEM((INTER, HIDDEN), jnp.bfloat16),
            pltpu.VMEM((INTER, HIDDEN), jnp.bfloat16),
            pltpu.VMEM((HIDDEN, INTER), jnp.bfloat16),
        ],
    )
    return pl.pallas_call(
        _ggemm_body,
        grid_spec=grid_spec,
        out_shape=jax.ShapeDtypeStruct((PADTOT, HIDDEN), jnp.float32),
        compiler_params=pltpu.CompilerParams(
            dimension_semantics=("arbitrary",)),
    )(bexp, xs, w1, w1, w2, gs)


def _combine(ys, posf):
    mesh = plsc.VectorSubcoreMesh(core_axis_name="c", subcore_axis_name="s")

    ch = TPW // 2

    @functools.partial(
        pl.kernel,
        out_type=jax.ShapeDtypeStruct((T, HIDDEN), jnp.float32),
        mesh=mesh,
        scratch_types=[
            pltpu.VMEM((TPW,), jnp.int32),
            pltpu.VMEM((TPW,), jnp.int32),
            pltpu.VMEM((ch, HIDDEN), jnp.float32),
            pltpu.VMEM((ch, HIDDEN), jnp.float32),
            pltpu.VMEM((ch, HIDDEN), jnp.float32),
            pltpu.SemaphoreType.DMA,
            pltpu.SemaphoreType.DMA,
            pltpu.SemaphoreType.DMA,
            pltpu.SemaphoreType.DMA,
        ],
    )
    def k(ys_hbm, posf_hbm, out_hbm, idx0_v, idx1_v, bufa, bufb, bufc,
          sa, sb, sc, so):
        wid = lax.axis_index("s") * 2 + lax.axis_index("c")
        base = wid * TPW
        pltpu.sync_copy(posf_hbm.at[pl.ds(base, TPW)], idx0_v)
        pltpu.sync_copy(posf_hbm.at[pl.ds(T + base, TPW)], idx1_v)

        def add_into(dst, src):
            def arow(i, _):
                def agrp(g, _):
                    plsc.addupdate(dst.at[i, pl.ds(g * 16, 16)],
                                   src[i, pl.ds(g * 16, 16)])
                    return 0
                lax.fori_loop(0, HIDDEN // 16, agrp, 0, unroll=8)
                return 0
            lax.fori_loop(0, ch, arow, 0)

        ca = pltpu.async_copy(ys_hbm.at[idx0_v.at[pl.ds(0, ch)]], bufa, sa)
        cb = pltpu.async_copy(ys_hbm.at[idx1_v.at[pl.ds(0, ch)]], bufb, sb)
        cc = pltpu.async_copy(ys_hbm.at[idx0_v.at[pl.ds(ch, ch)]], bufc, sc)
        ca.wait()
        cb.wait()
        add_into(bufa, bufb)
        st_a = pltpu.async_copy(bufa, out_hbm.at[pl.ds(base, ch)], so)
        cb2 = pltpu.async_copy(ys_hbm.at[idx1_v.at[pl.ds(ch, ch)]], bufb, sb)
        cc.wait()
        cb2.wait()
        add_into(bufc, bufb)
        st_c = pltpu.async_copy(bufc, out_hbm.at[pl.ds(base + ch, ch)], so)
        st_a.wait()
        st_c.wait()

    return k(ys, posf)


def kernel(hidden_states, gate_weight, w1, w2):
    x = hidden_states.reshape(T, HIDDEN)
    gwp = jnp.zeros((EPAD, HIDDEN), jnp.float32).at[:NUM_EXPERTS].set(
        gate_weight)

    pos2, g16, bexp_col = _router(x, gwp)
    posf = pos2.T.reshape(T * TOP_K)
    pos3 = posf.reshape(NTILE, NCH, SCH)
    bexp = bexp_col[:NBLK, 0]

    xs, gs = _dispatch(x, pos3, g16)
    ys = _ggemm(bexp, xs, gs, w1, w2)
    out = _combine(ys, posf)
    return out.reshape(1, T, HIDDEN)
